# baseline (speedup 1.0000x reference)
import functools

import jax
import jax.numpy as jnp
from jax import lax
from jax.experimental import pallas as pl
from jax.experimental.pallas import tpu as pltpu
from jax.experimental.pallas import tpu_sc as plsc

N = 10000
E = 320000
F = 128
NC, NS = 2, 16
NW = NC * NS
CH = 128
NCH = 79
E_PAD = NW * NCH * CH
NACC = 10016
Z_LO = 624
Z_HI = NACC - (NS - 1) * Z_LO
D_LO = 624
D_HI = N - (NS - 1) * D_LO
RCH = 104
RNCH = Z_LO // RCH
ZTAIL = Z_HI - RNCH * RCH
DTAIL = D_HI - RNCH * RCH


def _sc_mesh():
    return plsc.VectorSubcoreMesh(core_axis_name="c", subcore_axis_name="s",
                                  num_cores=NC, num_subcores=NS)


def _sc_degree(didx):
    @functools.partial(
        pl.kernel,
        out_type=jax.ShapeDtypeStruct((NC * N,), jnp.float32),
        mesh=_sc_mesh(),
        scratch_types=[
            pltpu.VMEM((NCH, CH), jnp.int32),
            pltpu.VMEM((CH,), jnp.float32),
            pltpu.VMEM((Z_HI,), jnp.float32),
            pltpu.VMEM_SHARED((NACC,), jnp.float32),
        ],
    )
    def k(didx_hbm, out_hbm, dall, ones, zbuf, acc):
        cid = lax.axis_index("c")
        sid = lax.axis_index("s")
        wid = sid * NC + cid
        one16 = jnp.ones((16,), jnp.float32)
        zero16 = jnp.zeros((16,), jnp.float32)
        for j in range(CH // 16):
            ones[pl.ds(j * 16, 16)] = one16
        for j in range(Z_HI // 16):
            zbuf[pl.ds(j * 16, 16)] = zero16

        @pl.when(sid < NS - 1)
        def _():
            pltpu.sync_copy(zbuf.at[pl.ds(0, Z_LO)],
                            acc.at[pl.ds(sid * Z_LO, Z_LO)])

        @pl.when(sid == NS - 1)
        def _():
            pltpu.sync_copy(zbuf, acc.at[pl.ds((NS - 1) * Z_LO, Z_HI)])

        plsc.subcore_barrier()
        pltpu.sync_copy(didx_hbm.at[wid], dall)

        def chunk(c, carry):
            pltpu.sync_copy(ones, acc.at[dall.at[c]], add=True)
            return carry

        lax.fori_loop(0, NCH, chunk, 0)
        plsc.subcore_barrier()

        @pl.when(sid < NS - 1)
        def _():
            pltpu.sync_copy(acc.at[pl.ds(sid * D_LO, D_LO)],
                            zbuf.at[pl.ds(0, D_LO)])
            pltpu.sync_copy(zbuf.at[pl.ds(0, D_LO)],
                            out_hbm.at[pl.ds(cid * N + sid * D_LO, D_LO)])

        @pl.when(sid == NS - 1)
        def _():
            pltpu.sync_copy(acc.at[pl.ds((NS - 1) * D_LO, D_HI)],
                            zbuf.at[pl.ds(0, D_HI)])
            pltpu.sync_copy(
                zbuf.at[pl.ds(0, D_HI)],
                out_hbm.at[pl.ds(cid * N + (NS - 1) * D_LO, D_HI)])

    return k(didx)


def _sc_scatter(g, idx):
    @functools.partial(
        pl.kernel,
        out_type=jax.ShapeDtypeStruct((NC, N, F), jnp.float32),
        mesh=_sc_mesh(),
        scratch_types=[
            pltpu.VMEM((2 * CH,), jnp.int32),
            pltpu.VMEM((2 * CH,), jnp.int32),
            pltpu.VMEM((2 * CH,), jnp.int32),
            pltpu.VMEM((2 * CH,), jnp.int32),
            pltpu.VMEM((CH, F), jnp.float32),
            pltpu.VMEM((CH, F), jnp.float32),
            pltpu.VMEM((RCH, F), jnp.float32),
            pltpu.VMEM_SHARED((NACC, F), jnp.float32),
            pltpu.SemaphoreType.DMA,
            pltpu.SemaphoreType.DMA,
            pltpu.SemaphoreType.DMA,
            pltpu.SemaphoreType.DMA,
            pltpu.SemaphoreType.DMA,
            pltpu.SemaphoreType.DMA,
        ],
    )
    def k(g_hbm, idx_hbm, out_hbm,
          ib0, ib1, ib2, ib3, rows0, rows1, zbuf, acc,
          is0, is1, is2, is3, gs0, gs1):
        cid = lax.axis_index("c")
        sid = lax.axis_index("s")
        wid = sid * NC + cid
        ibufs = (ib0, ib1, ib2, ib3)
        isems = (is0, is1, is2, is3)
        rowss = (rows0, rows1)
        gsems = (gs0, gs1)
        zero16 = jnp.zeros((16,), jnp.float32)

        def zrow(r, carry):
            for j in range(F // 16):
                zbuf[r, pl.ds(j * 16, 16)] = zero16
            return carry

        lax.fori_loop(0, RCH, zrow, 0)
        zrow0 = sid * Z_LO

        def zacc(kk, carry):
            pltpu.sync_copy(zbuf, acc.at[pl.ds(zrow0 + kk * RCH, RCH)])
            return carry

        lax.fori_loop(0, RNCH, zacc, 0)

        @pl.when(sid == NS - 1)
        def _():
            pltpu.sync_copy(zbuf.at[pl.ds(0, ZTAIL)],
                            acc.at[pl.ds(zrow0 + RNCH * RCH, ZTAIL)])

        plsc.subcore_barrier()
        pltpu.sync_copy(idx_hbm.at[wid, 0], ib0)
        pltpu.async_copy(g_hbm.at[ib0.at[pl.ds(0, CH)]], rows0, gs0)
        pltpu.sync_copy(idx_hbm.at[wid, 1], ib1)
        pltpu.async_copy(g_hbm.at[ib1.at[pl.ds(0, CH)]], rows1, gs1)
        pltpu.async_copy(idx_hbm.at[wid, 2], ib2, is2)
        pltpu.async_copy(idx_hbm.at[wid, 3], ib3, is3)

        def chunk(c, carry):
            for par in range(4):
                ibuf, isem = ibufs[par], isems[par]
                nbuf, nsem = ibufs[(par + 2) % 4], isems[(par + 2) % 4]
                rows, gsem = rowss[par % 2], gsems[par % 2]

                @pl.when(lax.rem(c, 4) == par)
                def _():
                    pltpu.make_async_copy(
                        g_hbm.at[ibuf.at[pl.ds(0, CH)]], rows, gsem).wait()
                    pltpu.sync_copy(rows, acc.at[ibuf.at[pl.ds(CH, CH)]],
                                    add=True)

                    @pl.when(c + 2 < NCH)
                    def _():
                        pltpu.make_async_copy(
                            idx_hbm.at[wid, c + 2], nbuf, nsem).wait()
                        pltpu.async_copy(
                            g_hbm.at[nbuf.at[pl.ds(0, CH)]], rows, gsem)

                    @pl.when(c + 4 < NCH)
                    def _():
                        pltpu.async_copy(idx_hbm.at[wid, c + 4], ibuf, isem)
            return carry

        lax.fori_loop(0, NCH, chunk, 0)
        plsc.subcore_barrier()
        drow0 = sid * D_LO

        def dissue(kk, carry):
            r = drow0 + kk * RCH
            pltpu.async_copy(acc.at[pl.ds(r, RCH)],
                             out_hbm.at[cid, pl.ds(r, RCH)], gs0)
            return carry

        lax.fori_loop(0, RNCH, dissue, 0)

        @pl.when(sid == NS - 1)
        def _():
            r = drow0 + RNCH * RCH
            pltpu.async_copy(acc.at[pl.ds(r, DTAIL)],
                             out_hbm.at[cid, pl.ds(r, DTAIL)], gs1)
            pltpu.make_async_copy(acc.at[pl.ds(r, DTAIL)],
                                  out_hbm.at[cid, pl.ds(r, DTAIL)], gs1).wait()

        def dwait(kk, carry):
            r = drow0 + kk * RCH
            pltpu.make_async_copy(acc.at[pl.ds(r, RCH)],
                                  out_hbm.at[cid, pl.ds(r, RCH)], gs0).wait()
            return carry

        lax.fori_loop(0, RNCH, dwait, 0)

    return k(g, idx)


BM = 1000


def _tc_pre(x, W, dinv):
    def body(x_ref, w_ref, d_ref, o_ref):
        o_ref[...] = d_ref[...] * jnp.dot(
            x_ref[...], w_ref[...], preferred_element_type=jnp.float32)

    return pl.pallas_call(
        body,
        grid=(N // BM,),
        in_specs=[pl.BlockSpec((BM, F), lambda i: (i, 0)),
                  pl.BlockSpec((F, F), lambda i: (0, 0)),
                  pl.BlockSpec((BM, 1), lambda i: (i, 0))],
        out_specs=pl.BlockSpec((BM, F), lambda i: (i, 0)),
        out_shape=jax.ShapeDtypeStruct((N, F), jnp.float32),
    )(x, W, dinv)


def _tc_mid(parts, g1, b1, W2, dinv):
    def body(p_ref, g_ref, b_ref, w_ref, d_ref, o_ref):
        d = d_ref[...]
        z = d * (p_ref[0] + p_ref[1] + g_ref[...]) + b_ref[...]
        z = jnp.maximum(z, 0.0)
        o_ref[...] = d * jnp.dot(z, w_ref[...],
                                 preferred_element_type=jnp.float32)

    return pl.pallas_call(
        body,
        grid=(N // BM,),
        in_specs=[pl.BlockSpec((NC, BM, F), lambda i: (0, i, 0)),
                  pl.BlockSpec((BM, F), lambda i: (i, 0)),
                  pl.BlockSpec((1, F), lambda i: (0, 0)),
                  pl.BlockSpec((F, F), lambda i: (0, 0)),
                  pl.BlockSpec((BM, 1), lambda i: (i, 0))],
        out_specs=pl.BlockSpec((BM, F), lambda i: (i, 0)),
        out_shape=jax.ShapeDtypeStruct((N, F), jnp.float32),
    )(parts, g1, b1.reshape(1, F), W2, dinv)


def _tc_post(parts, g2, b2, dinv):
    def body(p_ref, g_ref, b_ref, d_ref, o_ref):
        o_ref[...] = (d_ref[...] * (p_ref[0] + p_ref[1] + g_ref[...])
                      + b_ref[...])

    return pl.pallas_call(
        body,
        grid=(N // BM,),
        in_specs=[pl.BlockSpec((NC, BM, F), lambda i: (0, i, 0)),
                  pl.BlockSpec((BM, F), lambda i: (i, 0)),
                  pl.BlockSpec((1, F), lambda i: (0, 0)),
                  pl.BlockSpec((BM, 1), lambda i: (i, 0))],
        out_specs=pl.BlockSpec((BM, F), lambda i: (i, 0)),
        out_shape=jax.ShapeDtypeStruct((N, F), jnp.float32),
    )(parts, g2, b2.reshape(1, F), dinv)


def kernel(x, edge_index, edge_attr, y, W1, b1, W2, b2, We, be):
    pad = E_PAD - E
    pad_src = jnp.arange(pad, dtype=jnp.int32) % N
    pad_dst = N + (jnp.arange(pad, dtype=jnp.int32) % (NACC - N))
    sidx = jnp.concatenate(
        [edge_index[0], pad_src]).reshape(NW, NCH, CH)
    didx = jnp.concatenate(
        [edge_index[1], pad_dst]).reshape(NW, NCH, CH)
    idx = jnp.concatenate([sidx, didx], axis=-1)
    degp = _sc_degree(didx).reshape(NC, N)
    deg = degp[0] + degp[1] + 1.0
    dinv = lax.rsqrt(deg).reshape(N, 1)
    g1 = _tc_pre(x, W1, dinv)
    p1 = _sc_scatter(g1, idx)
    g2 = _tc_mid(p1, g1, b1, W2, dinv)
    p2 = _sc_scatter(g2, idx)
    return _tc_post(p2, g2, b2, dinv)
